# baseline (speedup 1.0000x reference)
import functools

import jax
import jax.numpy as jnp
from jax import lax
from jax.experimental import pallas as pl
from jax.experimental.pallas import tpu as pltpu
from jax.experimental.pallas import tpu_sc as plsc

_NC = 2
_NS = 16
_L = 16
_G = 64


def _sc_mesh():
    return plsc.VectorSubcoreMesh(
        core_axis_name="c", subcore_axis_name="s", num_cores=_NC, num_subcores=_NS
    )


def _make_deg_kernel(E, N, W):
    B = 40
    per_tile = E // (_NC * _NS)
    nb = per_tile // B
    STR = (N // _NS) // 8 * 8
    SPAN = N - STR * (_NS - 1)
    ZR = 80
    nz = SPAN // ZR
    assert per_tile % B == 0 and E % (_NC * _NS) == 0 and SPAN % ZR == 0
    assert nb % 2 == 1

    @functools.partial(
        pl.kernel,
        out_type=jax.ShapeDtypeStruct((2 * N, W), jnp.float32),
        mesh=_sc_mesh(),
        scratch_types=[
            pltpu.VMEM_SHARED((N, W), jnp.float32),
            pltpu.VMEM((ZR, W), jnp.float32),
            pltpu.VMEM((B, W), jnp.float32),
            pltpu.VMEM((B,), jnp.int32),
            pltpu.VMEM((B,), jnp.int32),
            pltpu.SemaphoreType.DMA,
            pltpu.SemaphoreType.DMA,
            pltpu.SemaphoreType.DMA,
            pltpu.SemaphoreType.DMA,
        ],
    )
    def deg_kernel(dst_hbm, out, acc, zbuf, ones_v, didx0, didx1,
                   isem0, isem1, ssem0, ssem1):
        cid = lax.axis_index("c")
        sid = lax.axis_index("s")
        zero16 = jnp.zeros((_L,), jnp.float32)
        one16 = jnp.ones((_L,), jnp.float32)

        def zb(i, carry):
            for c in range(W // _L):
                zbuf[i, pl.ds(c * _L, _L)] = zero16
            return carry

        lax.fori_loop(0, ZR, zb, 0)

        def ob(i, carry):
            for c in range(W // _L):
                ones_v[i, pl.ds(c * _L, _L)] = one16
            return carry

        lax.fori_loop(0, B, ob, 0)

        for k in range(nz):
            pltpu.sync_copy(zbuf, acc.at[pl.ds(sid * STR + k * ZR, ZR)])
        plsc.subcore_barrier()

        base = (cid * _NS + sid) * per_tile
        didx = (didx0, didx1)
        isem = (isem0, isem1)
        ssem = (ssem0, ssem1)

        def start_idx(j, b):
            pltpu.async_copy(dst_hbm.at[pl.ds(base + j * B, B)], didx[b], isem[b])

        def wait_idx(b):
            pltpu.make_async_copy(dst_hbm.at[pl.ds(0, B)], didx[b], isem[b]).wait()

        def start_scatter(b):
            pltpu.async_copy(ones_v, acc.at[didx[b]], ssem[b], add=True)

        def wait_scatter(b):
            pltpu.make_async_copy(ones_v, acc.at[didx[b]], ssem[b]).wait()

        start_idx(0, 0)
        start_idx(1, 1)

        def pair(g, carry):
            j0 = 2 * g
            wait_idx(0)
            start_scatter(0)
            wait_idx(1)
            start_scatter(1)
            wait_scatter(0)
            start_idx(j0 + 2, 0)
            wait_scatter(1)

            @pl.when(j0 + 3 < nb)
            def _():
                start_idx(j0 + 3, 1)

            return carry

        lax.fori_loop(0, (nb - 1) // 2, pair, 0)
        wait_idx(0)
        start_scatter(0)
        wait_scatter(0)
        plsc.subcore_barrier()

        sl = pl.ds(sid * STR, SPAN)
        pltpu.sync_copy(acc.at[sl], out.at[pl.ds(cid * N + sid * STR, SPAN)])

    return deg_kernel


def _make_agg_kernel(E, N, H):
    B = 80
    per_tile = E // _NS
    nb = per_tile // B
    STR = (N // _NS) // 8 * 8
    SPAN = N - STR * (_NS - 1)
    ZR = 80
    nz = SPAN // ZR
    assert per_tile % B == 0 and SPAN % ZR == 0 and H % _L == 0
    assert nb % 2 == 1

    @functools.partial(
        pl.kernel,
        out_type=(
            jax.ShapeDtypeStruct((N, H), jnp.float32),
            jax.ShapeDtypeStruct((N, H), jnp.float32),
        ),
        mesh=_sc_mesh(),
        scratch_types=[
            pltpu.VMEM_SHARED((N, H), jnp.float32),
            pltpu.VMEM((ZR, H), jnp.float32),
            pltpu.VMEM((B,), jnp.int32),
            pltpu.VMEM((B,), jnp.int32),
            pltpu.VMEM((B,), jnp.int32),
            pltpu.VMEM((B,), jnp.int32),
            pltpu.VMEM((B, H), jnp.float32),
            pltpu.VMEM((B, H), jnp.float32),
            pltpu.SemaphoreType.DMA,
            pltpu.SemaphoreType.DMA,
            pltpu.SemaphoreType.DMA,
            pltpu.SemaphoreType.DMA,
            pltpu.SemaphoreType.DMA,
            pltpu.SemaphoreType.DMA,
        ],
    )
    def agg_kernel(hlo, hhi, src_hbm, dst_hbm, out_lo, out_hi,
                   acc, zbuf, sidx0, sidx1, didx0, didx1, rows0, rows1,
                   isem0, isem1, gsem0, gsem1, ssem0, ssem1):
        cid = lax.axis_index("c")
        sid = lax.axis_index("s")
        zero16 = jnp.zeros((_L,), jnp.float32)

        def zb(i, carry):
            for c in range(H // _L):
                zbuf[i, pl.ds(c * _L, _L)] = zero16
            return carry

        lax.fori_loop(0, ZR, zb, 0)
        for k in range(nz):
            pltpu.sync_copy(zbuf, acc.at[pl.ds(sid * STR + k * ZR, ZR)])
        plsc.subcore_barrier()

        base = sid * per_tile
        sidx = (sidx0, sidx1)
        didx = (didx0, didx1)
        rows = (rows0, rows1)
        isem = (isem0, isem1)
        gsem = (gsem0, gsem1)
        ssem = (ssem0, ssem1)

        def start_idx(j, b):
            off = base + j * B
            pltpu.async_copy(src_hbm.at[pl.ds(off, B)], sidx[b], isem[b])
            pltpu.async_copy(dst_hbm.at[pl.ds(off, B)], didx[b], isem[b])

        def wait_idx(b):
            pltpu.make_async_copy(src_hbm.at[pl.ds(0, B)], sidx[b], isem[b]).wait()
            pltpu.make_async_copy(dst_hbm.at[pl.ds(0, B)], didx[b], isem[b]).wait()

        def run(table):
            def start_gather(b):
                pltpu.async_copy(table.at[sidx[b]], rows[b], gsem[b])

            def wait_gather(b):
                pltpu.make_async_copy(table.at[sidx[b]], rows[b], gsem[b]).wait()

            def start_scatter(b):
                pltpu.async_copy(rows[b], acc.at[didx[b]], ssem[b], add=True)

            def wait_scatter(b):
                pltpu.make_async_copy(rows[b], acc.at[didx[b]], ssem[b]).wait()

            start_idx(0, 0)
            start_idx(1, 1)
            wait_idx(0)
            start_gather(0)

            def pair(g, carry):
                j0 = 2 * g
                wait_gather(0)
                start_scatter(0)
                wait_idx(1)
                start_gather(1)
                wait_scatter(0)
                start_idx(j0 + 2, 0)
                wait_gather(1)
                start_scatter(1)
                wait_idx(0)
                start_gather(0)
                wait_scatter(1)

                @pl.when(j0 + 3 < nb)
                def _():
                    start_idx(j0 + 3, 1)

                return carry

            lax.fori_loop(0, (nb - 1) // 2, pair, 0)
            wait_gather(0)
            start_scatter(0)
            wait_scatter(0)

        @pl.when(cid == 0)
        def _():
            run(hlo)

        @pl.when(cid == 1)
        def _():
            run(hhi)

        plsc.subcore_barrier()
        sl = pl.ds(sid * STR, SPAN)

        @pl.when(cid == 0)
        def _():
            pltpu.sync_copy(acc.at[sl], out_lo.at[sl])

        @pl.when(cid == 1)
        def _():
            pltpu.sync_copy(acc.at[sl], out_hi.at[sl])

    return agg_kernel


def _tc_matmul(x, W1):
    N, _ = x.shape
    DH = W1.shape[1]

    def body(x_ref, w_ref, o_ref):
        o_ref[...] = jnp.dot(x_ref[...], w_ref[...],
                             preferred_element_type=jnp.float32)

    return pl.pallas_call(
        body, out_shape=jax.ShapeDtypeStruct((N, DH), jnp.float32)
    )(x, W1)


def _tc_scale(h1, degp):
    N, DH = h1.shape
    Hh = DH // 2

    def body(h_ref, d_ref, lo_ref, hi_ref, dinv_ref):
        lanes = d_ref.shape[1]
        cnt = jnp.sum(d_ref[:N], axis=1) + jnp.sum(d_ref[N:], axis=1)
        deg = cnt * (1.0 / lanes) + 1.0
        dinv = lax.rsqrt(deg)
        dinv_ref[...] = dinv
        hs = h_ref[...] * dinv[:, None]
        lo_ref[...] = hs[:, :Hh]
        hi_ref[...] = hs[:, Hh:]

    return pl.pallas_call(
        body,
        out_shape=(
            jax.ShapeDtypeStruct((N, Hh), jnp.float32),
            jax.ShapeDtypeStruct((N, Hh), jnp.float32),
            jax.ShapeDtypeStruct((N,), jnp.float32),
        ),
    )(h1, degp)


def _tc_mid(alo, ahi, hlo, hhi, dinv, b1, W2):
    N, Hh = alo.shape
    DH = W2.shape[1]

    def body(alo_ref, ahi_ref, hlo_ref, hhi_ref, dinv_ref, b_ref, w_ref,
             olo_ref, ohi_ref):
        dinv = dinv_ref[...]
        agg = jnp.concatenate([alo_ref[...], ahi_ref[...]], axis=1)
        hs = jnp.concatenate([hlo_ref[...], hhi_ref[...]], axis=1)
        z = (agg + hs) * dinv[:, None] + b_ref[...][None, :]
        z = jnp.maximum(z, 0.0)
        h2 = jnp.dot(z, w_ref[...], preferred_element_type=jnp.float32)
        h2s = h2 * dinv[:, None]
        olo_ref[...] = h2s[:, : DH // 2]
        ohi_ref[...] = h2s[:, DH // 2:]

    return pl.pallas_call(
        body,
        out_shape=(
            jax.ShapeDtypeStruct((N, DH // 2), jnp.float32),
            jax.ShapeDtypeStruct((N, DH // 2), jnp.float32),
        ),
    )(alo, ahi, hlo, hhi, dinv, b1, W2)


def _tc_post(alo, ahi, hlo, hhi, dinv, b2, batch, W_lin, b_lin):
    N, Hh = alo.shape
    DOUT = W_lin.shape[1]

    def body(alo_ref, ahi_ref, hlo_ref, hhi_ref, dinv_ref, b_ref, batch_ref,
             wl_ref, bl_ref, out_ref):
        dinv = dinv_ref[...]
        agg = jnp.concatenate([alo_ref[...], ahi_ref[...]], axis=1)
        hs = jnp.concatenate([hlo_ref[...], hhi_ref[...]], axis=1)
        h2 = (agg + hs) * dinv[:, None] + b_ref[...][None, :]
        seg = batch_ref[...]
        gids = lax.broadcasted_iota(jnp.int32, (_G, N), 0)
        onehot = (seg[None, :] == gids).astype(jnp.float32)
        sums = jnp.dot(onehot, h2, preferred_element_type=jnp.float32)
        cnt = jnp.sum(onehot, axis=1)
        pooled = sums / jnp.maximum(cnt, 1.0)[:, None]
        out_ref[...] = (
            jnp.dot(pooled, wl_ref[...], preferred_element_type=jnp.float32)
            + bl_ref[...][None, :]
        )

    return pl.pallas_call(
        body,
        out_shape=jax.ShapeDtypeStruct((_G, DOUT), jnp.float32),
    )(alo, ahi, hlo, hhi, dinv, b2, batch, W_lin, b_lin)


def kernel(x, edge_index, edge_attr, batch, W1, b1, W2, b2, W_lin, b_lin):
    del edge_attr
    N, _ = x.shape
    E = edge_index.shape[1]
    DH = W1.shape[1]
    src = edge_index[0]
    dst = edge_index[1]

    degp = _make_deg_kernel(E, N, DH // 2)(dst)
    h1 = _tc_matmul(x, W1)
    h1lo, h1hi, dinv = _tc_scale(h1, degp)
    agg = _make_agg_kernel(E, N, DH // 2)
    a1lo, a1hi = agg(h1lo, h1hi, src, dst)
    h2lo, h2hi = _tc_mid(a1lo, a1hi, h1lo, h1hi, dinv, b1, W2)
    a2lo, a2hi = agg(h2lo, h2hi, src, dst)
    return _tc_post(a2lo, a2hi, h2lo, h2hi, dinv, b2, batch, W_lin, b_lin)


# speedup vs baseline: 1.0501x; 1.0501x over previous
import functools

import jax
import jax.numpy as jnp
from jax import lax
from jax.experimental import pallas as pl
from jax.experimental.pallas import tpu as pltpu
from jax.experimental.pallas import tpu_sc as plsc

_NC = 2
_NS = 16
_L = 16
_G = 64


def _sc_mesh():
    return plsc.VectorSubcoreMesh(
        core_axis_name="c", subcore_axis_name="s", num_cores=_NC, num_subcores=_NS
    )


_BD = 40
_BA = 80


def _make_deg_kernel(E, N, W):
    B = _BD
    per_tile = E // (_NC * _NS)
    nb = per_tile // B
    STR = (N // _NS) // 8 * 8
    SPAN = N - STR * (_NS - 1)
    ZR = 80
    nz = SPAN // ZR
    assert per_tile % B == 0 and E % (_NC * _NS) == 0 and SPAN % ZR == 0
    assert nb % 2 == 1

    @functools.partial(
        pl.kernel,
        out_type=jax.ShapeDtypeStruct((2 * N, W), jnp.float32),
        mesh=_sc_mesh(),
        scratch_types=[
            pltpu.VMEM_SHARED((N, W), jnp.float32),
            pltpu.VMEM((ZR, W), jnp.float32),
            pltpu.VMEM((B, W), jnp.float32),
            pltpu.VMEM((nb, 1, B), jnp.int32),
            pltpu.SemaphoreType.DMA,
            pltpu.SemaphoreType.DMA,
            pltpu.SemaphoreType.DMA,
        ],
    )
    def deg_kernel(dst3_hbm, out, acc, zbuf, ones_v, didx3,
                   lsem, ssem0, ssem1):
        cid = lax.axis_index("c")
        sid = lax.axis_index("s")
        zero16 = jnp.zeros((_L,), jnp.float32)
        one16 = jnp.ones((_L,), jnp.float32)

        def zb(i, carry):
            for c in range(W // _L):
                zbuf[i, pl.ds(c * _L, _L)] = zero16
            return carry

        lax.fori_loop(0, ZR, zb, 0)

        def ob(i, carry):
            for c in range(W // _L):
                ones_v[i, pl.ds(c * _L, _L)] = one16
            return carry

        lax.fori_loop(0, B, ob, 0)

        row0 = (cid * _NS + sid) * nb
        pltpu.async_copy(dst3_hbm.at[pl.ds(row0, nb)], didx3, lsem).wait()

        for k in range(nz):
            pltpu.sync_copy(zbuf, acc.at[pl.ds(sid * STR + k * ZR, ZR)])
        plsc.subcore_barrier()

        ssem = (ssem0, ssem1)

        def start_scatter(j, b):
            pltpu.async_copy(ones_v, acc.at[didx3.at[j, 0]], ssem[b], add=True)

        def wait_scatter(b):
            pltpu.make_async_copy(ones_v, acc.at[didx3.at[0, 0]], ssem[b]).wait()

        start_scatter(0, 0)
        start_scatter(1, 1)

        def pair(g, carry):
            j0 = 2 * g
            wait_scatter(0)
            start_scatter(j0 + 2, 0)
            wait_scatter(1)

            @pl.when(j0 + 3 < nb)
            def _():
                start_scatter(j0 + 3, 1)

            return carry

        lax.fori_loop(0, (nb - 1) // 2, pair, 0)
        wait_scatter(0)
        plsc.subcore_barrier()

        sl = pl.ds(sid * STR, SPAN)
        pltpu.sync_copy(acc.at[sl], out.at[pl.ds(cid * N + sid * STR, SPAN)])

    return deg_kernel


def _make_agg_kernel(E, N, H):
    B = _BA
    per_tile = E // _NS
    nb = per_tile // B
    STR = (N // _NS) // 8 * 8
    SPAN = N - STR * (_NS - 1)
    ZR = 16
    nz = SPAN // ZR
    assert per_tile % B == 0 and SPAN % ZR == 0 and H % _L == 0
    assert nb % 2 == 1

    @functools.partial(
        pl.kernel,
        out_type=(
            jax.ShapeDtypeStruct((N, H), jnp.float32),
            jax.ShapeDtypeStruct((N, H), jnp.float32),
        ),
        mesh=_sc_mesh(),
        scratch_types=[
            pltpu.VMEM_SHARED((N, H), jnp.float32),
            pltpu.VMEM((ZR, H), jnp.float32),
            pltpu.VMEM((per_tile,), jnp.int32),
            pltpu.VMEM((nb, 1, B), jnp.int32),
            pltpu.VMEM((B, H), jnp.float32),
            pltpu.VMEM((B, H), jnp.float32),
            pltpu.SemaphoreType.DMA,
            pltpu.SemaphoreType.DMA,
            pltpu.SemaphoreType.DMA,
            pltpu.SemaphoreType.DMA,
            pltpu.SemaphoreType.DMA,
        ],
    )
    def agg_kernel(hlo, hhi, src_hbm, dst3_hbm, out_lo, out_hi,
                   acc, zbuf, sidx_all, didx3, rows0, rows1,
                   lsem, gsem0, gsem1, ssem0, ssem1):
        cid = lax.axis_index("c")
        sid = lax.axis_index("s")
        zero16 = jnp.zeros((_L,), jnp.float32)

        def zb(i, carry):
            for c in range(H // _L):
                zbuf[i, pl.ds(c * _L, _L)] = zero16
            return carry

        lax.fori_loop(0, ZR, zb, 0)
        pltpu.async_copy(src_hbm.at[pl.ds(sid * per_tile, per_tile)],
                         sidx_all, lsem)
        pltpu.async_copy(dst3_hbm.at[pl.ds(sid * nb, nb)], didx3, lsem)
        for k in range(nz):
            pltpu.sync_copy(zbuf, acc.at[pl.ds(sid * STR + k * ZR, ZR)])
        pltpu.make_async_copy(src_hbm.at[pl.ds(0, per_tile)], sidx_all, lsem).wait()
        pltpu.make_async_copy(dst3_hbm.at[pl.ds(0, nb)], didx3, lsem).wait()
        plsc.subcore_barrier()

        rows = (rows0, rows1)
        gsem = (gsem0, gsem1)
        ssem = (ssem0, ssem1)

        def run(table):
            def start_gather(j, b):
                pltpu.async_copy(
                    table.at[sidx_all.at[pl.ds(j * B, B)]], rows[b], gsem[b])

            def wait_gather(b):
                pltpu.make_async_copy(
                    table.at[sidx_all.at[pl.ds(0, B)]], rows[b], gsem[b]).wait()

            def start_scatter(j, b):
                pltpu.async_copy(rows[b], acc.at[didx3.at[j, 0]], ssem[b],
                                 add=True)

            def wait_scatter(b):
                pltpu.make_async_copy(rows[b], acc.at[didx3.at[0, 0]],
                                      ssem[b]).wait()

            start_gather(0, 0)

            def pair(g, carry):
                j0 = 2 * g
                wait_gather(0)
                start_gather(j0 + 1, 1)
                start_scatter(j0, 0)
                wait_gather(1)
                wait_scatter(0)
                start_gather(j0 + 2, 0)
                start_scatter(j0 + 1, 1)
                wait_scatter(1)
                return carry

            lax.fori_loop(0, (nb - 1) // 2, pair, 0)
            wait_gather(0)
            start_scatter(nb - 1, 0)
            wait_scatter(0)

        @pl.when(cid == 0)
        def _():
            run(hlo)

        @pl.when(cid == 1)
        def _():
            run(hhi)

        plsc.subcore_barrier()
        sl = pl.ds(sid * STR, SPAN)

        @pl.when(cid == 0)
        def _():
            pltpu.sync_copy(acc.at[sl], out_lo.at[sl])

        @pl.when(cid == 1)
        def _():
            pltpu.sync_copy(acc.at[sl], out_hi.at[sl])

    return agg_kernel


def _tc_matmul(x, W1):
    N, _ = x.shape
    DH = W1.shape[1]

    def body(x_ref, w_ref, o_ref):
        o_ref[...] = jnp.dot(x_ref[...], w_ref[...],
                             preferred_element_type=jnp.float32)

    return pl.pallas_call(
        body, out_shape=jax.ShapeDtypeStruct((N, DH), jnp.float32)
    )(x, W1)


def _tc_scale(h1, degp):
    N, DH = h1.shape
    Hh = DH // 2

    def body(h_ref, d_ref, lo_ref, hi_ref, dinv_ref):
        lanes = d_ref.shape[1]
        cnt = jnp.sum(d_ref[:N], axis=1) + jnp.sum(d_ref[N:], axis=1)
        deg = cnt * (1.0 / lanes) + 1.0
        dinv = lax.rsqrt(deg)
        dinv_ref[...] = dinv
        hs = h_ref[...] * dinv[:, None]
        lo_ref[...] = hs[:, :Hh]
        hi_ref[...] = hs[:, Hh:]

    return pl.pallas_call(
        body,
        out_shape=(
            jax.ShapeDtypeStruct((N, Hh), jnp.float32),
            jax.ShapeDtypeStruct((N, Hh), jnp.float32),
            jax.ShapeDtypeStruct((N,), jnp.float32),
        ),
    )(h1, degp)


def _tc_mid(alo, ahi, hlo, hhi, dinv, b1, W2):
    N, Hh = alo.shape
    DH = W2.shape[1]

    def body(alo_ref, ahi_ref, hlo_ref, hhi_ref, dinv_ref, b_ref, w_ref,
             olo_ref, ohi_ref):
        dinv = dinv_ref[...]
        agg = jnp.concatenate([alo_ref[...], ahi_ref[...]], axis=1)
        hs = jnp.concatenate([hlo_ref[...], hhi_ref[...]], axis=1)
        z = (agg + hs) * dinv[:, None] + b_ref[...][None, :]
        z = jnp.maximum(z, 0.0)
        h2 = jnp.dot(z, w_ref[...], preferred_element_type=jnp.float32)
        h2s = h2 * dinv[:, None]
        olo_ref[...] = h2s[:, : DH // 2]
        ohi_ref[...] = h2s[:, DH // 2:]

    return pl.pallas_call(
        body,
        out_shape=(
            jax.ShapeDtypeStruct((N, DH // 2), jnp.float32),
            jax.ShapeDtypeStruct((N, DH // 2), jnp.float32),
        ),
    )(alo, ahi, hlo, hhi, dinv, b1, W2)


def _tc_post(alo, ahi, hlo, hhi, dinv, b2, batch, W_lin, b_lin):
    N, Hh = alo.shape
    DOUT = W_lin.shape[1]

    def body(alo_ref, ahi_ref, hlo_ref, hhi_ref, dinv_ref, b_ref, batch_ref,
             wl_ref, bl_ref, out_ref):
        dinv = dinv_ref[...]
        agg = jnp.concatenate([alo_ref[...], ahi_ref[...]], axis=1)
        hs = jnp.concatenate([hlo_ref[...], hhi_ref[...]], axis=1)
        h2 = (agg + hs) * dinv[:, None] + b_ref[...][None, :]
        seg = batch_ref[...]
        gids = lax.broadcasted_iota(jnp.int32, (_G, N), 0)
        onehot = (seg[None, :] == gids).astype(jnp.float32)
        sums = jnp.dot(onehot, h2, preferred_element_type=jnp.float32)
        cnt = jnp.sum(onehot, axis=1)
        pooled = sums / jnp.maximum(cnt, 1.0)[:, None]
        out_ref[...] = (
            jnp.dot(pooled, wl_ref[...], preferred_element_type=jnp.float32)
            + bl_ref[...][None, :]
        )

    return pl.pallas_call(
        body,
        out_shape=jax.ShapeDtypeStruct((_G, DOUT), jnp.float32),
    )(alo, ahi, hlo, hhi, dinv, b2, batch, W_lin, b_lin)


def kernel(x, edge_index, edge_attr, batch, W1, b1, W2, b2, W_lin, b_lin):
    del edge_attr
    N, _ = x.shape
    E = edge_index.shape[1]
    DH = W1.shape[1]
    src = edge_index[0]
    dst = edge_index[1]

    dst3d = dst.reshape(E // _BD, 1, _BD)
    dst3a = dst.reshape(E // _BA, 1, _BA)
    degp = _make_deg_kernel(E, N, DH // 2)(dst3d)
    h1 = _tc_matmul(x, W1)
    h1lo, h1hi, dinv = _tc_scale(h1, degp)
    agg = _make_agg_kernel(E, N, DH // 2)
    a1lo, a1hi = agg(h1lo, h1hi, src, dst3a)
    h2lo, h2hi = _tc_mid(a1lo, a1hi, h1lo, h1hi, dinv, b1, W2)
    a2lo, a2hi = agg(h2lo, h2hi, src, dst3a)
    return _tc_post(a2lo, a2hi, h2lo, h2hi, dinv, b2, batch, W_lin, b_lin)


# speedup vs baseline: 1.0598x; 1.0093x over previous
import functools

import jax
import jax.numpy as jnp
from jax import lax
from jax.experimental import pallas as pl
from jax.experimental.pallas import tpu as pltpu
from jax.experimental.pallas import tpu_sc as plsc

_NC = 2
_NS = 16
_L = 16
_G = 64


def _sc_mesh():
    return plsc.VectorSubcoreMesh(
        core_axis_name="c", subcore_axis_name="s", num_cores=_NC, num_subcores=_NS
    )


_BD = 40
_BA = 80


def _make_deg_kernel(E, N, W):
    B = _BD
    per_tile = E // (_NC * _NS)
    nb = per_tile // B
    STR = (N // _NS) // 8 * 8
    SPAN = N - STR * (_NS - 1)
    ZR = 80
    nz = SPAN // ZR
    assert per_tile % B == 0 and E % (_NC * _NS) == 0 and SPAN % ZR == 0
    assert nb % 2 == 1

    @functools.partial(
        pl.kernel,
        out_type=jax.ShapeDtypeStruct((2 * N, W), jnp.float32),
        mesh=_sc_mesh(),
        compiler_params=pltpu.CompilerParams(use_tc_tiling_on_sc=False),
        scratch_types=[
            pltpu.VMEM_SHARED((N, W), jnp.float32),
            pltpu.VMEM((ZR, W), jnp.float32),
            pltpu.VMEM((B, W), jnp.float32),
            pltpu.VMEM((nb, 1, B), jnp.int32),
            pltpu.SemaphoreType.DMA,
            pltpu.SemaphoreType.DMA,
            pltpu.SemaphoreType.DMA,
        ],
    )
    def deg_kernel(dst3_hbm, out, acc, zbuf, ones_v, didx3,
                   lsem, ssem0, ssem1):
        cid = lax.axis_index("c")
        sid = lax.axis_index("s")
        zero16 = jnp.zeros((_L,), jnp.float32)
        one16 = jnp.ones((_L,), jnp.float32)

        def zb(i, carry):
            for c in range(W // _L):
                zbuf[i, pl.ds(c * _L, _L)] = zero16
            return carry

        lax.fori_loop(0, ZR, zb, 0)

        def ob(i, carry):
            for c in range(W // _L):
                ones_v[i, pl.ds(c * _L, _L)] = one16
            return carry

        lax.fori_loop(0, B, ob, 0)

        row0 = (cid * _NS + sid) * nb
        pltpu.async_copy(dst3_hbm.at[pl.ds(row0, nb)], didx3, lsem).wait()

        for k in range(nz):
            pltpu.sync_copy(zbuf, acc.at[pl.ds(sid * STR + k * ZR, ZR)])
        plsc.subcore_barrier()

        ssem = (ssem0, ssem1)

        def start_scatter(j, b):
            pltpu.async_copy(ones_v, acc.at[didx3.at[j, 0]], ssem[b], add=True)

        def wait_scatter(b):
            pltpu.make_async_copy(ones_v, acc.at[didx3.at[0, 0]], ssem[b]).wait()

        start_scatter(0, 0)
        start_scatter(1, 1)

        def pair(g, carry):
            j0 = 2 * g
            wait_scatter(0)
            start_scatter(j0 + 2, 0)
            wait_scatter(1)

            @pl.when(j0 + 3 < nb)
            def _():
                start_scatter(j0 + 3, 1)

            return carry

        lax.fori_loop(0, (nb - 1) // 2, pair, 0)
        wait_scatter(0)
        plsc.subcore_barrier()

        sl = pl.ds(sid * STR, SPAN)
        pltpu.sync_copy(acc.at[sl], out.at[pl.ds(cid * N + sid * STR, SPAN)])

    return deg_kernel


def _make_agg_kernel(E, N, H):
    B = _BA
    per_tile = E // _NS
    nb = per_tile // B
    STR = (N // _NS) // 8 * 8
    SPAN = N - STR * (_NS - 1)
    ZR = 16
    nz = SPAN // ZR
    assert per_tile % B == 0 and SPAN % ZR == 0 and H % _L == 0
    assert nb % 2 == 1

    @functools.partial(
        pl.kernel,
        out_type=(
            jax.ShapeDtypeStruct((N, H), jnp.float32),
            jax.ShapeDtypeStruct((N, H), jnp.float32),
        ),
        mesh=_sc_mesh(),
        scratch_types=[
            pltpu.VMEM_SHARED((N, H), jnp.float32),
            pltpu.VMEM((ZR, H), jnp.float32),
            pltpu.VMEM((per_tile,), jnp.int32),
            pltpu.VMEM((nb, 1, B), jnp.int32),
            pltpu.VMEM((B, H), jnp.float32),
            pltpu.VMEM((B, H), jnp.float32),
            pltpu.SemaphoreType.DMA,
            pltpu.SemaphoreType.DMA,
            pltpu.SemaphoreType.DMA,
            pltpu.SemaphoreType.DMA,
            pltpu.SemaphoreType.DMA,
        ],
    )
    def agg_kernel(hlo, hhi, src_hbm, dst3_hbm, out_lo, out_hi,
                   acc, zbuf, sidx_all, didx3, rows0, rows1,
                   lsem, gsem0, gsem1, ssem0, ssem1):
        cid = lax.axis_index("c")
        sid = lax.axis_index("s")
        zero16 = jnp.zeros((_L,), jnp.float32)

        def zb(i, carry):
            for c in range(H // _L):
                zbuf[i, pl.ds(c * _L, _L)] = zero16
            return carry

        lax.fori_loop(0, ZR, zb, 0)
        pltpu.async_copy(src_hbm.at[pl.ds(sid * per_tile, per_tile)],
                         sidx_all, lsem)
        pltpu.async_copy(dst3_hbm.at[pl.ds(sid * nb, nb)], didx3, lsem)
        for k in range(nz):
            pltpu.sync_copy(zbuf, acc.at[pl.ds(sid * STR + k * ZR, ZR)])
        pltpu.make_async_copy(src_hbm.at[pl.ds(0, per_tile)], sidx_all, lsem).wait()
        pltpu.make_async_copy(dst3_hbm.at[pl.ds(0, nb)], didx3, lsem).wait()
        plsc.subcore_barrier()

        rows = (rows0, rows1)
        gsem = (gsem0, gsem1)
        ssem = (ssem0, ssem1)

        def run(table):
            def start_gather(j, b):
                pltpu.async_copy(
                    table.at[sidx_all.at[pl.ds(j * B, B)]], rows[b], gsem[b])

            def wait_gather(b):
                pltpu.make_async_copy(
                    table.at[sidx_all.at[pl.ds(0, B)]], rows[b], gsem[b]).wait()

            def start_scatter(j, b):
                pltpu.async_copy(rows[b], acc.at[didx3.at[j, 0]], ssem[b],
                                 add=True)

            def wait_scatter(b):
                pltpu.make_async_copy(rows[b], acc.at[didx3.at[0, 0]],
                                      ssem[b]).wait()

            start_gather(0, 0)

            def pair(g, carry):
                j0 = 2 * g
                wait_gather(0)
                start_gather(j0 + 1, 1)
                start_scatter(j0, 0)
                wait_gather(1)
                wait_scatter(0)
                start_gather(j0 + 2, 0)
                start_scatter(j0 + 1, 1)
                wait_scatter(1)
                return carry

            lax.fori_loop(0, (nb - 1) // 2, pair, 0)
            wait_gather(0)
            start_scatter(nb - 1, 0)
            wait_scatter(0)

        @pl.when(cid == 0)
        def _():
            run(hlo)

        @pl.when(cid == 1)
        def _():
            run(hhi)

        plsc.subcore_barrier()
        sl = pl.ds(sid * STR, SPAN)

        @pl.when(cid == 0)
        def _():
            pltpu.sync_copy(acc.at[sl], out_lo.at[sl])

        @pl.when(cid == 1)
        def _():
            pltpu.sync_copy(acc.at[sl], out_hi.at[sl])

    return agg_kernel


def _tc_matmul(x, W1):
    N, _ = x.shape
    DH = W1.shape[1]

    def body(x_ref, w_ref, o_ref):
        o_ref[...] = jnp.dot(x_ref[...], w_ref[...],
                             preferred_element_type=jnp.float32)

    return pl.pallas_call(
        body, out_shape=jax.ShapeDtypeStruct((N, DH), jnp.float32)
    )(x, W1)


def _tc_scale(h1, degp):
    N, DH = h1.shape
    Hh = DH // 2

    def body(h_ref, d_ref, lo_ref, hi_ref, dinv_ref):
        lanes = d_ref.shape[1]
        cnt = jnp.sum(d_ref[:N], axis=1) + jnp.sum(d_ref[N:], axis=1)
        deg = cnt * (1.0 / lanes) + 1.0
        dinv = lax.rsqrt(deg)
        dinv_ref[...] = dinv
        hs = h_ref[...] * dinv[:, None]
        lo_ref[...] = hs[:, :Hh]
        hi_ref[...] = hs[:, Hh:]

    return pl.pallas_call(
        body,
        out_shape=(
            jax.ShapeDtypeStruct((N, Hh), jnp.float32),
            jax.ShapeDtypeStruct((N, Hh), jnp.float32),
            jax.ShapeDtypeStruct((N,), jnp.float32),
        ),
    )(h1, degp)


def _tc_mid(alo, ahi, hlo, hhi, dinv, b1, W2):
    N, Hh = alo.shape
    DH = W2.shape[1]

    def body(alo_ref, ahi_ref, hlo_ref, hhi_ref, dinv_ref, b_ref, w_ref,
             olo_ref, ohi_ref):
        dinv = dinv_ref[...]
        agg = jnp.concatenate([alo_ref[...], ahi_ref[...]], axis=1)
        hs = jnp.concatenate([hlo_ref[...], hhi_ref[...]], axis=1)
        z = (agg + hs) * dinv[:, None] + b_ref[...][None, :]
        z = jnp.maximum(z, 0.0)
        h2 = jnp.dot(z, w_ref[...], preferred_element_type=jnp.float32)
        h2s = h2 * dinv[:, None]
        olo_ref[...] = h2s[:, : DH // 2]
        ohi_ref[...] = h2s[:, DH // 2:]

    return pl.pallas_call(
        body,
        out_shape=(
            jax.ShapeDtypeStruct((N, DH // 2), jnp.float32),
            jax.ShapeDtypeStruct((N, DH // 2), jnp.float32),
        ),
    )(alo, ahi, hlo, hhi, dinv, b1, W2)


def _tc_post(alo, ahi, hlo, hhi, dinv, b2, batch, W_lin, b_lin):
    N, Hh = alo.shape
    DOUT = W_lin.shape[1]

    def body(alo_ref, ahi_ref, hlo_ref, hhi_ref, dinv_ref, b_ref, batch_ref,
             wl_ref, bl_ref, out_ref):
        dinv = dinv_ref[...]
        agg = jnp.concatenate([alo_ref[...], ahi_ref[...]], axis=1)
        hs = jnp.concatenate([hlo_ref[...], hhi_ref[...]], axis=1)
        h2 = (agg + hs) * dinv[:, None] + b_ref[...][None, :]
        seg = batch_ref[...]
        gids = lax.broadcasted_iota(jnp.int32, (_G, N), 0)
        onehot = (seg[None, :] == gids).astype(jnp.float32)
        sums = jnp.dot(onehot, h2, preferred_element_type=jnp.float32)
        cnt = jnp.sum(onehot, axis=1)
        pooled = sums / jnp.maximum(cnt, 1.0)[:, None]
        out_ref[...] = (
            jnp.dot(pooled, wl_ref[...], preferred_element_type=jnp.float32)
            + bl_ref[...][None, :]
        )

    return pl.pallas_call(
        body,
        out_shape=jax.ShapeDtypeStruct((_G, DOUT), jnp.float32),
    )(alo, ahi, hlo, hhi, dinv, b2, batch, W_lin, b_lin)


def kernel(x, edge_index, edge_attr, batch, W1, b1, W2, b2, W_lin, b_lin):
    del edge_attr
    N, _ = x.shape
    E = edge_index.shape[1]
    DH = W1.shape[1]
    src = edge_index[0]
    dst = edge_index[1]

    dst3d = dst.reshape(E // _BD, 1, _BD)
    dst3a = dst.reshape(E // _BA, 1, _BA)
    degp = _make_deg_kernel(E, N, _L)(dst3d)
    h1 = _tc_matmul(x, W1)
    h1lo, h1hi, dinv = _tc_scale(h1, degp)
    agg = _make_agg_kernel(E, N, DH // 2)
    a1lo, a1hi = agg(h1lo, h1hi, src, dst3a)
    h2lo, h2hi = _tc_mid(a1lo, a1hi, h1lo, h1hi, dinv, b1, W2)
    a2lo, a2hi = agg(h2lo, h2hi, src, dst3a)
    return _tc_post(a2lo, a2hi, h2lo, h2hi, dinv, b2, batch, W_lin, b_lin)
